# baseline (speedup 1.0000x reference)
import functools

import jax
import jax.numpy as jnp
from jax import lax
from jax.experimental import pallas as pl
from jax.experimental.pallas import tpu as pltpu
import jax.experimental.pallas.tpu_sc as plsc

_ROWS = 65536
_D = 6 * 7 * 8
_HEAD = 8
_BLOCK = 8192
_LANE = 16


@functools.partial(
    pl.kernel,
    out_type=jax.ShapeDtypeStruct((_HEAD, _D), jnp.float32),
    mesh=plsc.VectorSubcoreMesh(core_axis_name="c", subcore_axis_name="s",
                                num_cores=1, num_subcores=1),
    scratch_types=[
        pltpu.VMEM((_HEAD, _D), jnp.float32),
    ],
)
def _sc_head(x_hbm, head_hbm, buf):
    pltpu.sync_copy(x_hbm.at[pl.ds(0, _HEAD)], buf)
    for k in range(2 * _D // _LANE):
        r, col = divmod(k * _LANE, _D)
        vec = (jnp.arange(_LANE, dtype=jnp.int32)
               + k * _LANE).astype(jnp.float32)
        buf[r, pl.ds(col, _LANE)] = vec
    pltpu.sync_copy(buf, head_hbm)


def _tc_copy(x_ref, o_ref):
    o_ref[...] = x_ref[...]


def _tc_patch(o_in_ref, head_ref, o_ref):
    o_ref[...] = head_ref[...]


def kernel(x):
    xf = x.reshape(_ROWS, _D)
    head = _sc_head(xf)
    bulk = pl.pallas_call(
        _tc_copy,
        grid=(_ROWS // _BLOCK,),
        in_specs=[pl.BlockSpec((_BLOCK, _D), lambda i: (i, 0))],
        out_specs=pl.BlockSpec((_BLOCK, _D), lambda i: (i, 0)),
        out_shape=jax.ShapeDtypeStruct((_ROWS, _D), jnp.float32),
    )(xf)
    out = pl.pallas_call(
        _tc_patch,
        grid=(1,),
        in_specs=[
            pl.BlockSpec(memory_space=pltpu.MemorySpace.HBM),
            pl.BlockSpec((_HEAD, _D), lambda i: (0, 0)),
        ],
        out_specs=pl.BlockSpec((_HEAD, _D), lambda i: (0, 0)),
        out_shape=jax.ShapeDtypeStruct((_ROWS, _D), jnp.float32),
        input_output_aliases={0: 0},
    )(bulk, head)
    return out.reshape(x.shape)
